# baseline (speedup 1.0000x reference)
import functools

import jax
import jax.numpy as jnp
from jax import lax
from jax.experimental import pallas as pl
from jax.experimental.pallas import tpu as pltpu
from jax.experimental.pallas import tpu_sc as plsc

N_NODES = 10000
DIM = 256
HALF = 128
N_EDGES = 160000
NC = 2
NS = 16
CHUNK = 128
E_PAD = 163840
N_ACC = N_NODES + 8
ROWS_A = 632
ROWS_B = N_NODES - 15 * ROWS_A
BM = 1000

_mesh = plsc.VectorSubcoreMesh(core_axis_name="c", subcore_axis_name="s")


def _copy_rows(src_ref, dst_ref, sid):
    @pl.when(sid < NS - 1)
    def _():
        row0 = pl.multiple_of(sid * ROWS_A, 8)
        pltpu.sync_copy(src_ref.at[pl.ds(row0, ROWS_A)],
                        dst_ref.at[pl.ds(row0, ROWS_A)])

    @pl.when(sid == NS - 1)
    def _():
        row0 = (NS - 1) * ROWS_A
        pltpu.sync_copy(src_ref.at[pl.ds(row0, ROWS_B)],
                        dst_ref.at[pl.ds(row0, ROWS_B)])


DEGW = 128


def _deg_body(dst_hbm, ones_hbm, zeros_hbm, out0, out1, acc, didx, ones_v, sem):
    del sem
    cid = lax.axis_index("c")
    sid = lax.axis_index("s")
    _copy_rows(zeros_hbm, acc, sid)
    pltpu.sync_copy(ones_hbm, ones_v)
    plsc.subcore_barrier()
    tile_base = cid * (E_PAD // NC) + sid * (E_PAD // (NC * NS))

    @pl.loop(0, E_PAD // (NC * NS * CHUNK))
    def _chunks(i):
        base = pl.multiple_of(tile_base + i * CHUNK, CHUNK)
        pltpu.sync_copy(dst_hbm.at[pl.ds(base, CHUNK)], didx)
        pltpu.sync_copy(ones_v, acc.at[didx], add=True)

    plsc.subcore_barrier()

    @pl.when(cid == 0)
    def _():
        _copy_rows(acc, out0, sid)

    @pl.when(cid == 1)
    def _():
        _copy_rows(acc, out1, sid)


_deg_call = functools.partial(
    pl.kernel,
    _deg_body,
    out_type=(
        jax.ShapeDtypeStruct((N_NODES, DEGW), jnp.float32),
        jax.ShapeDtypeStruct((N_NODES, DEGW), jnp.float32),
    ),
    mesh=_mesh,
    scratch_types=[
        pltpu.VMEM_SHARED((N_ACC, DEGW), jnp.float32),
        pltpu.VMEM((CHUNK,), jnp.int32),
        pltpu.VMEM((CHUNK, DEGW), jnp.float32),
        pltpu.SemaphoreType.DMA,
    ],
)()


PCH = 128
EPT = E_PAD // NS
NCH = EPT // PCH
NBUF = 3


def _prop_half(hs, out, src_hbm, dst_hbm, acc,
               rows, sidx, didx, gsem, ssem, isem, dsem, sid):
    _copy_rows(hs, acc, sid)
    plsc.subcore_barrier()
    ebase = pl.multiple_of(sid * EPT, CHUNK)

    def idx_hbm(ref, c):
        return ref.at[pl.ds(pl.multiple_of(ebase + c * PCH, PCH), PCH)]

    def fire_sidx(c, b):
        pltpu.async_copy(idx_hbm(src_hbm, c), sidx[b], isem[b])

    def wait_sidx(c, b):
        pltpu.make_async_copy(idx_hbm(src_hbm, c), sidx[b], isem[b]).wait()

    def fire_didx(c, b):
        pltpu.async_copy(idx_hbm(dst_hbm, c), didx[b], dsem[b])

    def wait_didx(c, b):
        pltpu.make_async_copy(idx_hbm(dst_hbm, c), didx[b], dsem[b]).wait()

    def fire_gather(c, b):
        pltpu.async_copy(hs.at[sidx[b]], rows[b], gsem[b])

    def wait_gather(c, b):
        pltpu.make_async_copy(hs.at[sidx[b]], rows[b], gsem[b]).wait()

    def fire_scatter(c, b):
        pltpu.async_copy(rows[b], acc.at[didx[b]], ssem[b], add=True)

    def wait_scatter(c, b):
        pltpu.make_async_copy(rows[b], acc.at[didx[b]], ssem[b]).wait()

    fire_sidx(0, 0)
    fire_sidx(1, 1)
    fire_sidx(2, 2)
    fire_didx(0, 0)
    wait_sidx(0, 0)
    fire_gather(0, 0)

    @pl.loop(0, (NCH + 2 + NBUF) // NBUF)
    def _steps(i):
        for k in range(NBUF):
            b = k
            f = (k + 1) % NBUF
            c = i * NBUF + k

            @pl.when(jnp.logical_and(c >= 2, c < NCH + 2))
            def _():
                wait_scatter(c - 2, f)

            @pl.when(c + 1 < NCH)
            def _():
                fire_didx(c + 1, f)
                wait_sidx(c + 1, f)
                fire_gather(c + 1, f)

            @pl.when(c < NCH)
            def _():
                wait_gather(c, b)

                @pl.when(c + 3 < NCH)
                def _():
                    fire_sidx(c + 3, b)

                wait_didx(c, b)
                fire_scatter(c, b)

    plsc.subcore_barrier()
    _copy_rows(acc, out, sid)


def _prop_body(hs0, hs1, src_hbm, dst_hbm, out0, out1, acc,
               r0, r1, r2, j0, j1, j2, i0, i1, i2,
               g0, g1, g2, s0, s1, s2, e0, e1, e2, d0, d1, d2):
    cid = lax.axis_index("c")
    sid = lax.axis_index("s")
    rows = (r0, r1, r2)
    sidx = (j0, j1, j2)
    didx = (i0, i1, i2)
    gsem = (g0, g1, g2)
    ssem = (s0, s1, s2)
    isem = (e0, e1, e2)
    dsem = (d0, d1, d2)

    @pl.when(cid == 0)
    def _():
        _prop_half(hs0, out0, src_hbm, dst_hbm, acc,
                   rows, sidx, didx, gsem, ssem, isem, dsem, sid)

    @pl.when(cid == 1)
    def _():
        _prop_half(hs1, out1, src_hbm, dst_hbm, acc,
                   rows, sidx, didx, gsem, ssem, isem, dsem, sid)


_prop_call = functools.partial(
    pl.kernel,
    _prop_body,
    out_type=(
        jax.ShapeDtypeStruct((N_NODES, HALF), jnp.float32),
        jax.ShapeDtypeStruct((N_NODES, HALF), jnp.float32),
    ),
    mesh=_mesh,
    scratch_types=[
        pltpu.VMEM_SHARED((N_ACC, HALF), jnp.float32),
    ]
    + [pltpu.VMEM((PCH, HALF), jnp.float32)] * NBUF
    + [pltpu.VMEM((PCH,), jnp.int32)] * (2 * NBUF)
    + [pltpu.SemaphoreType.DMA] * (4 * NBUF),
)()


def _dis(d0_ref, d1_ref):
    deg = d0_ref[...] + d1_ref[...] + 1.0
    return lax.rsqrt(deg)


def _mm1r_body(x_ref, w_ref, h0_ref, h1_ref):
    h = jnp.dot(x_ref[...], w_ref[...], preferred_element_type=jnp.float32)
    h0_ref[...] = h[:, :HALF]
    h1_ref[...] = h[:, HALF:]


def _scale_body(d0, d1, h0, h1, hs0_ref, hs1_ref):
    dis = _dis(d0, d1)
    hs0_ref[...] = h0[...] * dis
    hs1_ref[...] = h1[...] * dis


def _mm2_body(d0, d1, p0, p1, b_ref, w_ref, q0_ref, q1_ref):
    dis = _dis(d0, d1)
    p = jnp.concatenate([p0[...], p1[...]], axis=1)
    act = jnp.maximum(p * dis + b_ref[...], 0.0)
    h = jnp.dot(act, w_ref[...], preferred_element_type=jnp.float32)
    hs = h * dis
    q0_ref[...] = hs[:, :HALF]
    q1_ref[...] = hs[:, HALF:]


def _final_body(d0, d1, q0, q1, b_ref, out_ref):
    dis = _dis(d0, d1)
    q = jnp.concatenate([q0[...], q1[...]], axis=1)
    out_ref[...] = q * dis + b_ref[...]


def _row_spec(width):
    return pl.BlockSpec((BM, width), lambda i: (i, 0))


def _full_spec(shape):
    return pl.BlockSpec(shape, lambda i: (0,) * len(shape))


_GRID = (N_NODES // BM,)

_mm1r_call = pl.pallas_call(
    _mm1r_body,
    grid=_GRID,
    in_specs=[_row_spec(DIM), _full_spec((DIM, DIM))],
    out_specs=[_row_spec(HALF), _row_spec(HALF)],
    out_shape=[
        jax.ShapeDtypeStruct((N_NODES, HALF), jnp.float32),
        jax.ShapeDtypeStruct((N_NODES, HALF), jnp.float32),
    ],
)

_scale_call = pl.pallas_call(
    _scale_body,
    grid=_GRID,
    in_specs=[_row_spec(1), _row_spec(1), _row_spec(HALF), _row_spec(HALF)],
    out_specs=[_row_spec(HALF), _row_spec(HALF)],
    out_shape=[
        jax.ShapeDtypeStruct((N_NODES, HALF), jnp.float32),
        jax.ShapeDtypeStruct((N_NODES, HALF), jnp.float32),
    ],
)

_mm2_call = pl.pallas_call(
    _mm2_body,
    grid=_GRID,
    in_specs=[_row_spec(1), _row_spec(1), _row_spec(HALF), _row_spec(HALF),
              _full_spec((1, DIM)), _full_spec((DIM, DIM))],
    out_specs=[_row_spec(HALF), _row_spec(HALF)],
    out_shape=[
        jax.ShapeDtypeStruct((N_NODES, HALF), jnp.float32),
        jax.ShapeDtypeStruct((N_NODES, HALF), jnp.float32),
    ],
)

_final_call = pl.pallas_call(
    _final_body,
    grid=_GRID,
    in_specs=[_row_spec(1), _row_spec(1), _row_spec(HALF), _row_spec(HALF),
              _full_spec((1, DIM))],
    out_specs=_row_spec(DIM),
    out_shape=jax.ShapeDtypeStruct((N_NODES, DIM), jnp.float32),
)


def kernel(x, edge_index, W1, b1, W2, b2):
    src = edge_index[0].astype(jnp.int32)
    dst = edge_index[1].astype(jnp.int32)
    pad = E_PAD - N_EDGES
    src_p = jnp.concatenate([src, jnp.zeros((pad,), jnp.int32)])
    dst_p = jnp.concatenate([dst, jnp.full((pad,), N_NODES, jnp.int32)])
    ones = jnp.ones((CHUNK, DEGW), jnp.float32)
    zeros = jnp.zeros((N_NODES, DEGW), jnp.float32)

    deg0w, deg1w = _deg_call(dst_p, ones, zeros)
    h0, h1 = _mm1r_call(x, W1)
    deg0 = deg0w[:, :1]
    deg1 = deg1w[:, :1]
    b1r = b1.reshape(1, DIM)
    b2r = b2.reshape(1, DIM)

    hs0, hs1 = _scale_call(deg0, deg1, h0, h1)
    p0, p1 = _prop_call(hs0, hs1, src_p, dst_p)
    q0, q1 = _mm2_call(deg0, deg1, p0, p1, b1r, W2)
    r0, r1 = _prop_call(q0, q1, src_p, dst_p)
    return _final_call(deg0, deg1, r0, r1, b2r)


# speedup vs baseline: 1.4967x; 1.4967x over previous
import functools

import jax
import jax.numpy as jnp
from jax import lax
from jax.experimental import pallas as pl
from jax.experimental.pallas import tpu as pltpu
from jax.experimental.pallas import tpu_sc as plsc

N_NODES = 10000
DIM = 256
HALF = 128
N_EDGES = 160000
NC = 2
NS = 16
CHUNK = 128
E_PAD = 163840
N_ACC = N_NODES + 8
ROWS_A = 632
ROWS_B = N_NODES - 15 * ROWS_A
BM = 1000

_mesh = plsc.VectorSubcoreMesh(core_axis_name="c", subcore_axis_name="s")


def _copy_rows(src_ref, dst_ref, sid):
    @pl.when(sid < NS - 1)
    def _():
        row0 = pl.multiple_of(sid * ROWS_A, 8)
        pltpu.sync_copy(src_ref.at[pl.ds(row0, ROWS_A)],
                        dst_ref.at[pl.ds(row0, ROWS_A)])

    @pl.when(sid == NS - 1)
    def _():
        row0 = (NS - 1) * ROWS_A
        pltpu.sync_copy(src_ref.at[pl.ds(row0, ROWS_B)],
                        dst_ref.at[pl.ds(row0, ROWS_B)])


DEGW = 128


def _deg_body(dst_hbm, ones_hbm, zeros_hbm, out0, out1, acc, didx, ones_v, sem):
    del sem
    cid = lax.axis_index("c")
    sid = lax.axis_index("s")
    _copy_rows(zeros_hbm, acc, sid)
    pltpu.sync_copy(ones_hbm, ones_v)
    plsc.subcore_barrier()
    tile_base = cid * (E_PAD // NC) + sid * (E_PAD // (NC * NS))

    @pl.loop(0, E_PAD // (NC * NS * CHUNK))
    def _chunks(i):
        base = pl.multiple_of(tile_base + i * CHUNK, CHUNK)
        pltpu.sync_copy(dst_hbm.at[pl.ds(base, CHUNK)], didx)
        pltpu.sync_copy(ones_v, acc.at[didx], add=True)

    plsc.subcore_barrier()

    @pl.when(cid == 0)
    def _():
        _copy_rows(acc, out0, sid)

    @pl.when(cid == 1)
    def _():
        _copy_rows(acc, out1, sid)


_deg_call = functools.partial(
    pl.kernel,
    _deg_body,
    out_type=(
        jax.ShapeDtypeStruct((N_NODES, DEGW), jnp.float32),
        jax.ShapeDtypeStruct((N_NODES, DEGW), jnp.float32),
    ),
    mesh=_mesh,
    scratch_types=[
        pltpu.VMEM_SHARED((N_ACC, DEGW), jnp.float32),
        pltpu.VMEM((CHUNK,), jnp.int32),
        pltpu.VMEM((CHUNK, DEGW), jnp.float32),
        pltpu.SemaphoreType.DMA,
    ],
)()


PCH = 128
E_PROP = 161792
EPT = E_PROP // NS
NCH = EPT // PCH
NBUF = 3


def _prop_half(hs, out, src_hbm, dst_hbm, acc,
               rows, sidx, didx, gsem, ssem, isem, dsem, sid):
    _copy_rows(hs, acc, sid)
    plsc.subcore_barrier()
    ebase = pl.multiple_of(sid * EPT, CHUNK)

    def idx_hbm(ref, c):
        return ref.at[pl.ds(pl.multiple_of(ebase + c * PCH, PCH), PCH)]

    def fire_sidx(c, b):
        pltpu.async_copy(idx_hbm(src_hbm, c), sidx[b], isem[b])

    def wait_sidx(c, b):
        pltpu.make_async_copy(idx_hbm(src_hbm, c), sidx[b], isem[b]).wait()

    def fire_didx(c, b):
        pltpu.async_copy(idx_hbm(dst_hbm, c), didx[b], dsem[b])

    def wait_didx(c, b):
        pltpu.make_async_copy(idx_hbm(dst_hbm, c), didx[b], dsem[b]).wait()

    def fire_gather(c, b):
        pltpu.async_copy(hs.at[sidx[b]], rows[b], gsem[b])

    def wait_gather(c, b):
        pltpu.make_async_copy(hs.at[sidx[b]], rows[b], gsem[b]).wait()

    def fire_scatter(c, b):
        pltpu.async_copy(rows[b], acc.at[didx[b]], ssem[b], add=True)

    def wait_scatter(c, b):
        pltpu.make_async_copy(rows[b], acc.at[didx[b]], ssem[b]).wait()

    fire_sidx(0, 0)
    fire_sidx(1, 1)
    fire_sidx(2, 2)
    fire_didx(0, 0)
    wait_sidx(0, 0)
    fire_gather(0, 0)

    @pl.loop(0, (NCH + 2 + NBUF) // NBUF)
    def _steps(i):
        for k in range(NBUF):
            b = k
            f = (k + 1) % NBUF
            c = i * NBUF + k

            @pl.when(jnp.logical_and(c >= 2, c < NCH + 2))
            def _():
                wait_scatter(c - 2, f)

            @pl.when(c + 1 < NCH)
            def _():
                fire_didx(c + 1, f)
                wait_sidx(c + 1, f)
                fire_gather(c + 1, f)

            @pl.when(c < NCH)
            def _():
                wait_gather(c, b)

                @pl.when(c + 3 < NCH)
                def _():
                    fire_sidx(c + 3, b)

                wait_didx(c, b)
                fire_scatter(c, b)

    plsc.subcore_barrier()
    _copy_rows(acc, out, sid)


def _prop_body(hs0, hs1, src_hbm, dst_hbm, out0, out1, acc,
               r0, r1, r2, j0, j1, j2, i0, i1, i2,
               g0, g1, g2, s0, s1, s2, e0, e1, e2, d0, d1, d2):
    cid = lax.axis_index("c")
    sid = lax.axis_index("s")
    rows = (r0, r1, r2)
    sidx = (j0, j1, j2)
    didx = (i0, i1, i2)
    gsem = (g0, g1, g2)
    ssem = (s0, s1, s2)
    isem = (e0, e1, e2)
    dsem = (d0, d1, d2)

    @pl.when(cid == 0)
    def _():
        _prop_half(hs0, out0, src_hbm, dst_hbm, acc,
                   rows, sidx, didx, gsem, ssem, isem, dsem, sid)

    @pl.when(cid == 1)
    def _():
        _prop_half(hs1, out1, src_hbm, dst_hbm, acc,
                   rows, sidx, didx, gsem, ssem, isem, dsem, sid)


_prop_call = functools.partial(
    pl.kernel,
    _prop_body,
    out_type=(
        jax.ShapeDtypeStruct((N_NODES, HALF), jnp.float32),
        jax.ShapeDtypeStruct((N_NODES, HALF), jnp.float32),
    ),
    mesh=_mesh,
    scratch_types=[
        pltpu.VMEM_SHARED((N_ACC, HALF), jnp.float32),
    ]
    + [pltpu.VMEM((PCH, HALF), jnp.float32)] * NBUF
    + [pltpu.VMEM((PCH,), jnp.int32)] * (2 * NBUF)
    + [pltpu.SemaphoreType.DMA] * (4 * NBUF),
)()


def _dis(d0_ref, d1_ref):
    deg = d0_ref[...] + d1_ref[...] + 1.0
    return lax.rsqrt(deg)


def _mm1_body(d0, d1, x_ref, w_ref, hs0_ref, hs1_ref):
    dis = _dis(d0, d1)
    h = jnp.dot(x_ref[...], w_ref[...], preferred_element_type=jnp.float32)
    hs = h * dis
    hs0_ref[...] = hs[:, :HALF]
    hs1_ref[...] = hs[:, HALF:]


def _mm2_body(d0, d1, p0, p1, b_ref, w_ref, q0_ref, q1_ref):
    dis = _dis(d0, d1)
    p = jnp.concatenate([p0[...], p1[...]], axis=1)
    act = jnp.maximum(p * dis + b_ref[...], 0.0)
    h = jnp.dot(act, w_ref[...], preferred_element_type=jnp.float32)
    hs = h * dis
    q0_ref[...] = hs[:, :HALF]
    q1_ref[...] = hs[:, HALF:]


def _final_body(d0, d1, q0, q1, b_ref, out_ref):
    dis = _dis(d0, d1)
    q = jnp.concatenate([q0[...], q1[...]], axis=1)
    out_ref[...] = q * dis + b_ref[...]


def _row_spec(width):
    return pl.BlockSpec((BM, width), lambda i: (i, 0))


def _full_spec(shape):
    return pl.BlockSpec(shape, lambda i: (0,) * len(shape))


_GRID = (N_NODES // BM,)

_mm1_call = pl.pallas_call(
    _mm1_body,
    grid=_GRID,
    in_specs=[_row_spec(1), _row_spec(1), _row_spec(DIM), _full_spec((DIM, DIM))],
    out_specs=[_row_spec(HALF), _row_spec(HALF)],
    out_shape=[
        jax.ShapeDtypeStruct((N_NODES, HALF), jnp.float32),
        jax.ShapeDtypeStruct((N_NODES, HALF), jnp.float32),
    ],
)

_mm2_call = pl.pallas_call(
    _mm2_body,
    grid=_GRID,
    in_specs=[_row_spec(1), _row_spec(1), _row_spec(HALF), _row_spec(HALF),
              _full_spec((1, DIM)), _full_spec((DIM, DIM))],
    out_specs=[_row_spec(HALF), _row_spec(HALF)],
    out_shape=[
        jax.ShapeDtypeStruct((N_NODES, HALF), jnp.float32),
        jax.ShapeDtypeStruct((N_NODES, HALF), jnp.float32),
    ],
)

_final_call = pl.pallas_call(
    _final_body,
    grid=_GRID,
    in_specs=[_row_spec(1), _row_spec(1), _row_spec(HALF), _row_spec(HALF),
              _full_spec((1, DIM))],
    out_specs=_row_spec(DIM),
    out_shape=jax.ShapeDtypeStruct((N_NODES, DIM), jnp.float32),
)


def kernel(x, edge_index, W1, b1, W2, b2):
    src = edge_index[0].astype(jnp.int32)
    dst = edge_index[1].astype(jnp.int32)
    pad = E_PAD - N_EDGES
    src_p = jnp.concatenate([src, jnp.zeros((pad,), jnp.int32)])
    dst_p = jnp.concatenate([dst, jnp.full((pad,), N_NODES, jnp.int32)])
    ones = jnp.ones((CHUNK, DEGW), jnp.float32)
    zeros = jnp.zeros((N_NODES, DEGW), jnp.float32)

    deg0w, deg1w = _deg_call(dst_p, ones, zeros)
    deg0 = deg0w[:, :1]
    deg1 = deg1w[:, :1]
    b1r = b1.reshape(1, DIM)
    b2r = b2.reshape(1, DIM)

    hs0, hs1 = _mm1_call(deg0, deg1, x, W1)
    p0, p1 = _prop_call(hs0, hs1, src_p, dst_p)
    q0, q1 = _mm2_call(deg0, deg1, p0, p1, b1r, W2)
    r0, r1 = _prop_call(q0, q1, src_p, dst_p)
    return _final_call(deg0, deg1, r0, r1, b2r)


# speedup vs baseline: 1.5507x; 1.0361x over previous
import functools

import jax
import jax.numpy as jnp
from jax import lax
from jax.experimental import pallas as pl
from jax.experimental.pallas import tpu as pltpu
from jax.experimental.pallas import tpu_sc as plsc

N_NODES = 10000
DIM = 256
HALF = 128
N_EDGES = 160000
NC = 2
NS = 16
CHUNK = 128
E_PAD = 161792
N_ACC = N_NODES + 8
ROWS_A = 632
ROWS_B = N_NODES - 15 * ROWS_A
BM = 1000

_mesh = plsc.VectorSubcoreMesh(core_axis_name="c", subcore_axis_name="s")


def _copy_rows(src_ref, dst_ref, sid):
    @pl.when(sid < NS - 1)
    def _():
        row0 = pl.multiple_of(sid * ROWS_A, 8)
        pltpu.sync_copy(src_ref.at[pl.ds(row0, ROWS_A)],
                        dst_ref.at[pl.ds(row0, ROWS_A)])

    @pl.when(sid == NS - 1)
    def _():
        row0 = (NS - 1) * ROWS_A
        pltpu.sync_copy(src_ref.at[pl.ds(row0, ROWS_B)],
                        dst_ref.at[pl.ds(row0, ROWS_B)])


DEGW = 128
NC_DEG = N_EDGES // CHUNK
DEG_BASE = NC_DEG // (NC * NS)
DEG_EXTRA = NC_DEG - NC * NS * DEG_BASE


def _deg_body(dst_hbm, ones_hbm, zeros_hbm, out0, out1, acc,
              d0, d1, d2, ones_v, i0, i1, i2, s0, s1, s2):
    cid = lax.axis_index("c")
    sid = lax.axis_index("s")
    didx = (d0, d1, d2)
    isem = (i0, i1, i2)
    ssem = (s0, s1, s2)
    _copy_rows(zeros_hbm, acc, sid)
    pltpu.sync_copy(ones_hbm, ones_v)
    plsc.subcore_barrier()
    lid = cid * NS + sid
    nch = DEG_BASE + jnp.where(lid < DEG_EXTRA, 1, 0)
    cbase = lid * DEG_BASE + jnp.minimum(lid, DEG_EXTRA)

    def idx_hbm(c):
        return dst_hbm.at[pl.ds(pl.multiple_of((cbase + c) * CHUNK, CHUNK),
                                CHUNK)]

    def fire_didx(c, b):
        pltpu.async_copy(idx_hbm(c), didx[b], isem[b])

    def wait_didx(c, b):
        pltpu.make_async_copy(idx_hbm(c), didx[b], isem[b]).wait()

    def fire_scatter(c, b):
        pltpu.async_copy(ones_v, acc.at[didx[b]], ssem[b], add=True)

    def wait_scatter(c, b):
        pltpu.make_async_copy(ones_v, acc.at[didx[b]], ssem[b]).wait()

    fire_didx(0, 0)

    @pl.loop(0, (DEG_BASE + 1 + 2 + 3) // 3)
    def _steps(i):
        for k in range(3):
            b = k
            f = (k + 1) % 3
            c = i * 3 + k

            @pl.when(jnp.logical_and(c >= 2, c < nch + 2))
            def _():
                wait_scatter(c - 2, f)

            @pl.when(c + 1 < nch)
            def _():
                fire_didx(c + 1, f)

            @pl.when(c < nch)
            def _():
                wait_didx(c, b)
                fire_scatter(c, b)

    plsc.subcore_barrier()

    @pl.when(cid == 0)
    def _():
        _copy_rows(acc, out0, sid)

    @pl.when(cid == 1)
    def _():
        _copy_rows(acc, out1, sid)


_deg_call = functools.partial(
    pl.kernel,
    _deg_body,
    out_type=(
        jax.ShapeDtypeStruct((N_NODES, DEGW), jnp.float32),
        jax.ShapeDtypeStruct((N_NODES, DEGW), jnp.float32),
    ),
    mesh=_mesh,
    scratch_types=[
        pltpu.VMEM_SHARED((N_ACC, DEGW), jnp.float32),
    ]
    + [pltpu.VMEM((CHUNK,), jnp.int32)] * 3
    + [
        pltpu.VMEM((CHUNK, DEGW), jnp.float32),
    ]
    + [pltpu.SemaphoreType.DMA] * 6,
)()


PCH = 128
EPT = E_PAD // NS
NCH = EPT // PCH
NBUF = 3


def _prop_half(hs, out, src_hbm, dst_hbm, acc,
               rows, sidx, didx, gsem, ssem, isem, dsem, sid):
    _copy_rows(hs, acc, sid)
    plsc.subcore_barrier()
    ebase = pl.multiple_of(sid * EPT, CHUNK)

    def idx_hbm(ref, c):
        return ref.at[pl.ds(pl.multiple_of(ebase + c * PCH, PCH), PCH)]

    def fire_sidx(c, b):
        pltpu.async_copy(idx_hbm(src_hbm, c), sidx[b], isem[b])

    def wait_sidx(c, b):
        pltpu.make_async_copy(idx_hbm(src_hbm, c), sidx[b], isem[b]).wait()

    def fire_didx(c, b):
        pltpu.async_copy(idx_hbm(dst_hbm, c), didx[b], dsem[b])

    def wait_didx(c, b):
        pltpu.make_async_copy(idx_hbm(dst_hbm, c), didx[b], dsem[b]).wait()

    def fire_gather(c, b):
        pltpu.async_copy(hs.at[sidx[b]], rows[b], gsem[b])

    def wait_gather(c, b):
        pltpu.make_async_copy(hs.at[sidx[b]], rows[b], gsem[b]).wait()

    def fire_scatter(c, b):
        pltpu.async_copy(rows[b], acc.at[didx[b]], ssem[b], add=True)

    def wait_scatter(c, b):
        pltpu.make_async_copy(rows[b], acc.at[didx[b]], ssem[b]).wait()

    fire_sidx(0, 0)
    fire_sidx(1, 1)
    fire_sidx(2, 2)
    fire_didx(0, 0)
    wait_sidx(0, 0)
    fire_gather(0, 0)

    @pl.loop(0, (NCH + 2 + NBUF) // NBUF)
    def _steps(i):
        for k in range(NBUF):
            b = k
            f = (k + 1) % NBUF
            c = i * NBUF + k

            @pl.when(jnp.logical_and(c >= 2, c < NCH + 2))
            def _():
                wait_scatter(c - 2, f)

            @pl.when(c + 1 < NCH)
            def _():
                fire_didx(c + 1, f)
                wait_sidx(c + 1, f)
                fire_gather(c + 1, f)

            @pl.when(c < NCH)
            def _():
                wait_gather(c, b)

                @pl.when(c + 3 < NCH)
                def _():
                    fire_sidx(c + 3, b)

                wait_didx(c, b)
                fire_scatter(c, b)

    plsc.subcore_barrier()
    _copy_rows(acc, out, sid)


def _prop_body(hs0, hs1, src_hbm, dst_hbm, out0, out1, acc,
               r0, r1, r2, j0, j1, j2, i0, i1, i2,
               g0, g1, g2, s0, s1, s2, e0, e1, e2, d0, d1, d2):
    cid = lax.axis_index("c")
    sid = lax.axis_index("s")
    rows = (r0, r1, r2)
    sidx = (j0, j1, j2)
    didx = (i0, i1, i2)
    gsem = (g0, g1, g2)
    ssem = (s0, s1, s2)
    isem = (e0, e1, e2)
    dsem = (d0, d1, d2)

    @pl.when(cid == 0)
    def _():
        _prop_half(hs0, out0, src_hbm, dst_hbm, acc,
                   rows, sidx, didx, gsem, ssem, isem, dsem, sid)

    @pl.when(cid == 1)
    def _():
        _prop_half(hs1, out1, src_hbm, dst_hbm, acc,
                   rows, sidx, didx, gsem, ssem, isem, dsem, sid)


_prop_call = functools.partial(
    pl.kernel,
    _prop_body,
    out_type=(
        jax.ShapeDtypeStruct((N_NODES, HALF), jnp.float32),
        jax.ShapeDtypeStruct((N_NODES, HALF), jnp.float32),
    ),
    mesh=_mesh,
    scratch_types=[
        pltpu.VMEM_SHARED((N_ACC, HALF), jnp.float32),
    ]
    + [pltpu.VMEM((PCH, HALF), jnp.float32)] * NBUF
    + [pltpu.VMEM((PCH,), jnp.int32)] * (2 * NBUF)
    + [pltpu.SemaphoreType.DMA] * (4 * NBUF),
)()


def _dis(d0_ref, d1_ref):
    deg = d0_ref[...] + d1_ref[...] + 1.0
    return lax.rsqrt(deg)


def _mm1_body(d0, d1, x_ref, w_ref, hs0_ref, hs1_ref):
    dis = _dis(d0, d1)
    h = jnp.dot(x_ref[...], w_ref[...], preferred_element_type=jnp.float32)
    hs = h * dis
    hs0_ref[...] = hs[:, :HALF]
    hs1_ref[...] = hs[:, HALF:]


def _mm2_body(d0, d1, p0, p1, b_ref, w_ref, q0_ref, q1_ref):
    dis = _dis(d0, d1)
    p = jnp.concatenate([p0[...], p1[...]], axis=1)
    act = jnp.maximum(p * dis + b_ref[...], 0.0)
    h = jnp.dot(act, w_ref[...], preferred_element_type=jnp.float32)
    hs = h * dis
    q0_ref[...] = hs[:, :HALF]
    q1_ref[...] = hs[:, HALF:]


def _final_body(d0, d1, q0, q1, b_ref, out_ref):
    dis = _dis(d0, d1)
    q = jnp.concatenate([q0[...], q1[...]], axis=1)
    out_ref[...] = q * dis + b_ref[...]


def _row_spec(width):
    return pl.BlockSpec((BM, width), lambda i: (i, 0))


def _full_spec(shape):
    return pl.BlockSpec(shape, lambda i: (0,) * len(shape))


_GRID = (N_NODES // BM,)

_mm1_call = pl.pallas_call(
    _mm1_body,
    grid=_GRID,
    in_specs=[_row_spec(1), _row_spec(1), _row_spec(DIM), _full_spec((DIM, DIM))],
    out_specs=[_row_spec(HALF), _row_spec(HALF)],
    out_shape=[
        jax.ShapeDtypeStruct((N_NODES, HALF), jnp.float32),
        jax.ShapeDtypeStruct((N_NODES, HALF), jnp.float32),
    ],
)

_mm2_call = pl.pallas_call(
    _mm2_body,
    grid=_GRID,
    in_specs=[_row_spec(1), _row_spec(1), _row_spec(HALF), _row_spec(HALF),
              _full_spec((1, DIM)), _full_spec((DIM, DIM))],
    out_specs=[_row_spec(HALF), _row_spec(HALF)],
    out_shape=[
        jax.ShapeDtypeStruct((N_NODES, HALF), jnp.float32),
        jax.ShapeDtypeStruct((N_NODES, HALF), jnp.float32),
    ],
)

_final_call = pl.pallas_call(
    _final_body,
    grid=_GRID,
    in_specs=[_row_spec(1), _row_spec(1), _row_spec(HALF), _row_spec(HALF),
              _full_spec((1, DIM))],
    out_specs=_row_spec(DIM),
    out_shape=jax.ShapeDtypeStruct((N_NODES, DIM), jnp.float32),
)


def kernel(x, edge_index, W1, b1, W2, b2):
    src = edge_index[0].astype(jnp.int32)
    dst = edge_index[1].astype(jnp.int32)
    pad = E_PAD - N_EDGES
    src_p = jnp.concatenate([src, jnp.zeros((pad,), jnp.int32)])
    dst_p = jnp.concatenate([dst, jnp.full((pad,), N_NODES, jnp.int32)])
    ones = jnp.ones((CHUNK, DEGW), jnp.float32)
    zeros = jnp.zeros((N_NODES, DEGW), jnp.float32)

    deg0w, deg1w = _deg_call(dst_p, ones, zeros)
    deg0 = deg0w[:, :1]
    deg1 = deg1w[:, :1]
    b1r = b1.reshape(1, DIM)
    b2r = b2.reshape(1, DIM)

    hs0, hs1 = _mm1_call(deg0, deg1, x, W1)
    p0, p1 = _prop_call(hs0, hs1, src_p, dst_p)
    q0, q1 = _mm2_call(deg0, deg1, p0, p1, b1r, W2)
    r0, r1 = _prop_call(q0, q1, src_p, dst_p)
    return _final_call(deg0, deg1, r0, r1, b2r)
